# Initial kernel scaffold; baseline (speedup 1.0000x reference)
import jax
import jax.numpy as jnp
from jax.experimental import pallas as pl


def kernel(positions, ray_indices, alpha, occs):
    raise NotImplementedError("write your pallas kernel here")


# speedup vs baseline: 7.1877x; 7.1877x over previous
import functools

import jax
import jax.numpy as jnp
import numpy as np
from jax import lax
from jax.experimental import pallas as pl
from jax.experimental.pallas import tpu as pltpu
from jax.experimental.pallas import tpu_sc as plsc

RES = 128
N_RAYS = 4096
N_PTS = 2097152
ROWS = N_PTS // 128
BR = 512
NC, NS, L = 2, 16, 16
NW = NC * NS
PTS_PER_W = N_PTS // NW
CHUNK = 8192
NCHUNK = PTS_PER_W // CHUNK
RSL = N_RAYS // NS


def _prep_body(pos_ref, alpha_ref, occs_ref, w_ref, table_ref, vidx_ref):
    a = alpha_ref[...]
    av = jnp.clip(a, 1e-06, 1.0 - 1e-06)
    ent = -av * jnp.log(av) - (1.0 - av) * jnp.log(1.0 - av)
    surf = occs_ref[...] > 0.8
    val = jnp.where(surf & (a <= 0.0), 0.0, ent)
    bits = lax.bitcast_convert_type(val, jnp.int32)
    table_ref[...] = jnp.where(surf, bits | jnp.int32(-(2**31)), bits)
    f = jnp.minimum(jnp.floor(pos_ref[...] * RES), RES - 1)
    vidx_f = jnp.dot(f.astype(jnp.bfloat16), w_ref[...],
                     preferred_element_type=jnp.float32)
    vidx_ref[...] = vidx_f.astype(jnp.int32)


_prep = pl.pallas_call(
    _prep_body,
    grid=(ROWS // BR,),
    in_specs=[
        pl.BlockSpec((BR, 3 * 128), lambda i: (i, 0)),
        pl.BlockSpec((BR, 128), lambda i: (i, 0)),
        pl.BlockSpec((BR, 128), lambda i: (i, 0)),
        pl.BlockSpec((3 * 128, 128), lambda i: (0, 0)),
    ],
    out_specs=[
        pl.BlockSpec((BR, 128), lambda i: (i, 0)),
        pl.BlockSpec((BR, 128), lambda i: (i, 0)),
    ],
    out_shape=[
        jax.ShapeDtypeStruct((ROWS, 128), jnp.int32),
        jax.ShapeDtypeStruct((ROWS, 128), jnp.int32),
    ],
)

_mesh = plsc.VectorSubcoreMesh(core_axis_name="c", subcore_axis_name="s")


@functools.partial(
    pl.kernel,
    mesh=_mesh,
    compiler_params=pltpu.CompilerParams(needs_layout_passes=False),
    out_type=(
        jax.ShapeDtypeStruct((N_PTS,), jnp.int32),
        jax.ShapeDtypeStruct((NC, N_RAYS), jnp.float32),
    ),
    scratch_types=[
        pltpu.VMEM((CHUNK,), jnp.int32),
        pltpu.VMEM((CHUNK,), jnp.int32),
        pltpu.VMEM((CHUNK,), jnp.int32),
        pltpu.VMEM((N_RAYS,), jnp.float32),
        pltpu.VMEM((RSL,), jnp.float32),
        pltpu.VMEM((RSL,), jnp.float32),
        pltpu.VMEM_SHARED((NS, N_RAYS), jnp.float32),
        pltpu.SemaphoreType.DMA,
    ],
)
def _gather_flags(vidx_hbm, table_hbm, ray_hbm, g_hbm, flags_hbm,
                  idx_v, g_v, ray_v, fl_v, acc_v, tmp_v, fl_sh, sem):
    c = lax.axis_index("c")
    s = lax.axis_index("s")
    wid = c * NS + s
    base = wid * PTS_PER_W

    zero16 = jnp.zeros((L,), jnp.float32)

    def zf(i, carry):
        fl_v[pl.ds(i * L, L)] = zero16
        return carry

    lax.fori_loop(0, N_RAYS // L, zf, 0)

    ones = jnp.ones((L,), jnp.float32)

    def chunk_body(k, carry):
        off = base + k * CHUNK
        pltpu.sync_copy(vidx_hbm.at[pl.ds(off, CHUNK)], idx_v)
        pltpu.async_copy(table_hbm.at[idx_v], g_v, sem).wait()
        pltpu.sync_copy(ray_hbm.at[pl.ds(off, CHUNK)], ray_v)
        pltpu.sync_copy(g_v, g_hbm.at[pl.ds(off, CHUNK)])

        def vec_body(j, carry2):
            gi = g_v[pl.ds(j * L, L)]
            rv = ray_v[pl.ds(j * L, L)]
            plsc.store_scatter(fl_v, [rv], ones, mask=gi < 0)
            return carry2

        lax.fori_loop(0, CHUNK // L, vec_body, 0)
        return carry

    lax.fori_loop(0, NCHUNK, chunk_body, 0)

    pltpu.sync_copy(fl_v, fl_sh.at[s])
    plsc.subcore_barrier()
    pltpu.sync_copy(fl_sh.at[0, pl.ds(s * RSL, RSL)], acc_v)

    def red_body(t, carry):
        pltpu.sync_copy(fl_sh.at[t, pl.ds(s * RSL, RSL)], tmp_v)

        def add_body(j, carry2):
            acc_v[pl.ds(j * L, L)] = acc_v[pl.ds(j * L, L)] + tmp_v[pl.ds(j * L, L)]
            return carry2

        lax.fori_loop(0, RSL // L, add_body, 0)
        return carry

    lax.fori_loop(1, NS, red_body, 0)
    pltpu.sync_copy(acc_v, flags_hbm.at[c, pl.ds(s * RSL, RSL)])


@functools.partial(
    pl.kernel,
    mesh=_mesh,
    compiler_params=pltpu.CompilerParams(needs_layout_passes=False),
    out_type=jax.ShapeDtypeStruct((N_PTS,), jnp.float32),
    scratch_types=[
        pltpu.VMEM((CHUNK,), jnp.int32),
        pltpu.VMEM((CHUNK,), jnp.int32),
        pltpu.VMEM((CHUNK,), jnp.float32),
        pltpu.VMEM((N_RAYS,), jnp.float32),
        pltpu.VMEM((N_RAYS,), jnp.float32),
    ],
)
def _finalize(g_hbm, ray_hbm, flags_hbm, out_hbm, g_v, ray_v, o_v, fl_v, fl2_v):
    c = lax.axis_index("c")
    s = lax.axis_index("s")
    wid = c * NS + s
    base = wid * PTS_PER_W

    pltpu.sync_copy(flags_hbm.at[0], fl_v)
    pltpu.sync_copy(flags_hbm.at[1], fl2_v)

    def add_body(j, carry):
        fl_v[pl.ds(j * L, L)] = fl_v[pl.ds(j * L, L)] + fl2_v[pl.ds(j * L, L)]
        return carry

    lax.fori_loop(0, N_RAYS // L, add_body, 0)

    mag_mask = jnp.full((L,), 0x7FFFFFFF, jnp.int32)
    zero16 = jnp.zeros((L,), jnp.float32)

    def chunk_body(k, carry):
        off = base + k * CHUNK
        pltpu.sync_copy(g_hbm.at[pl.ds(off, CHUNK)], g_v)
        pltpu.sync_copy(ray_hbm.at[pl.ds(off, CHUNK)], ray_v)

        def vec_body(j, carry2):
            gi = g_v[pl.ds(j * L, L)]
            rv = ray_v[pl.ds(j * L, L)]
            mag = plsc.bitcast(gi & mag_mask, jnp.float32)
            fr = plsc.load_gather(fl_v, [rv])
            keep = (gi < 0) | (fr <= 0.0)
            o_v[pl.ds(j * L, L)] = jnp.where(keep, mag, zero16)
            return carry2

        lax.fori_loop(0, CHUNK // L, vec_body, 0)
        pltpu.sync_copy(o_v, out_hbm.at[pl.ds(off, CHUNK)])
        return carry

    lax.fori_loop(0, NCHUNK, chunk_body, 0)


def _make_w():
    w = np.zeros((3 * 128, 128), np.float32)
    j = np.arange(128)
    w[3 * j, j] = RES * RES
    w[3 * j + 1, j] = RES
    w[3 * j + 2, j] = 1.0
    return w


_W = _make_w()


def kernel(positions, ray_indices, alpha, occs):
    pos2 = positions.reshape(ROWS, 3 * 128)
    alpha2 = alpha.reshape(ROWS, 128)
    occs2 = occs.reshape(ROWS, 128)
    table, vidx = _prep(pos2, alpha2, occs2, jnp.asarray(_W, jnp.bfloat16))
    g, flags = _gather_flags(vidx.reshape(-1), table.reshape(-1), ray_indices)
    return _finalize(g, ray_indices, flags)
